# Initial kernel scaffold; baseline (speedup 1.0000x reference)
import jax
import jax.numpy as jnp
from jax.experimental import pallas as pl


def kernel(x, edge_index, edge_attr, batch, x_lin_W, x_lin_b, edge_table, W1, b1, W2, b2, bn_g, bn_b, lin1_W, lin1_b, lin2_W, lin2_b):
    raise NotImplementedError("write your pallas kernel here")


# speedup vs baseline: 5.0189x; 5.0189x over previous
import jax
import jax.numpy as jnp
from jax import lax
from jax.experimental import pallas as pl
from jax.experimental.pallas import tpu as pltpu
from jax.experimental.pallas import tpu_sc as plsc

_N = 10000
_E = 320000
_HID = 128
_G = 64
_NC = 2
_NS = 16
_C = 80

_NPAD = 10240
_RPT = _NPAD // _NS
_EPC = _E // _NC
_EPT = _EPC // _NS
_NCHUNK = _EPT // _C


def _edge_body(t_hbm, gidx_hbm, dst_hbm, zeros_hbm, out_hbm,
               idx_v, dst_v, rows_v, agg_sh, sem):
    c = lax.axis_index("c")
    s = lax.axis_index("s")
    pltpu.sync_copy(zeros_hbm, agg_sh.at[pl.ds(s * _RPT, _RPT)])
    plsc.subcore_barrier()
    base = c * _EPC + s * _EPT

    def chunk(k, carry):
        off = base + k * _C
        pltpu.sync_copy(gidx_hbm.at[pl.ds(off, _C)], idx_v)
        pltpu.sync_copy(dst_hbm.at[pl.ds(off, _C)], dst_v)
        pltpu.async_copy(t_hbm.at[idx_v], rows_v, sem).wait()
        pltpu.sync_copy(rows_v, agg_sh.at[dst_v], add=True)
        return carry

    lax.fori_loop(0, _NCHUNK, chunk, 0)
    plsc.subcore_barrier()
    pltpu.sync_copy(agg_sh.at[pl.ds(s * _RPT, _RPT)],
                    out_hbm.at[c, pl.ds(s * _RPT, _RPT)])


_edge_pass_cache = []


def _edge_pass(*args):
    if not _edge_pass_cache:
        _edge_pass_cache.append(pl.kernel(
            _edge_body,
            out_type=jax.ShapeDtypeStruct((_NC, _NPAD, _HID), jnp.float32),
            mesh=plsc.VectorSubcoreMesh(core_axis_name="c",
                                        subcore_axis_name="s"),
            scratch_types=[
                pltpu.VMEM((_C,), jnp.int32),
                pltpu.VMEM((_C,), jnp.int32),
                pltpu.VMEM((_C, _HID), jnp.float32),
                pltpu.VMEM_SHARED((_NPAD, _HID), jnp.float32),
                pltpu.SemaphoreType.DMA,
            ],
        ))
    return _edge_pass_cache[0](*args)


def _dot3(a, b, scr, dn=None):
    f = jnp.float32
    ah = a.astype(jnp.bfloat16)
    al = (a - ah.astype(f)).astype(jnp.bfloat16)
    bh = b.astype(jnp.bfloat16)
    bl = (b - bh.astype(f)).astype(jnp.bfloat16)
    if dn is None:
        dn = (((a.ndim - 1,), (0,)), ((), ()))
    dot = lambda u, v: lax.dot_general(u, v, dn, preferred_element_type=f)
    scr[...] = dot(al, bh)
    return (dot(ah, bl) + scr[...]) + dot(ah, bh)


def _tables(h, et_ref, t_ref):
    for a in range(4):
        t_ref[pl.ds(a, 1), :, :] = jnp.maximum(h + et_ref[pl.ds(a, 1), :], 0.0)[None]


def _encode_body(x_ref, w_ref, b_ref, et_ref, h_ref, t_ref, scr):
    h = _dot3(x_ref[...], w_ref[...], scr)
    h = h + b_ref[...]
    h_ref[...] = h
    _tables(h, et_ref, t_ref)


_encode = pl.pallas_call(
    _encode_body,
    out_shape=[jax.ShapeDtypeStruct((_N, _HID), jnp.float32),
               jax.ShapeDtypeStruct((4, _N, _HID), jnp.float32)],
    scratch_shapes=[pltpu.VMEM((_N, _HID), jnp.float32)],
)


def _mlp_body(with_tables, h_ref, a_ref, w1_ref, b1_ref, w2_ref,
              b2_ref, g_ref, bb_ref, et_ref, hn_ref, t_ref, scr):
    z = h_ref[...] + a_ref[...]
    z = jnp.maximum(_dot3(z, w1_ref[...], scr) + b1_ref[...], 0.0)
    z = jnp.maximum(_dot3(z, w2_ref[...], scr) + b2_ref[...], 0.0)
    m = jnp.mean(z, axis=0, keepdims=True)
    zc = z - m
    v = jnp.mean(zc * zc, axis=0, keepdims=True)
    hn = zc * lax.rsqrt(v + 1e-5) * g_ref[...] + bb_ref[...]
    hn_ref[...] = hn
    if with_tables:
        _tables(hn, et_ref, t_ref)


def _mlp_mid_body(h_ref, a_ref, w1_ref, b1_ref, w2_ref, b2_ref,
                  g_ref, bb_ref, et_ref, hn_ref, t_ref, scr):
    _mlp_body(True, h_ref, a_ref, w1_ref, b1_ref, w2_ref, b2_ref,
              g_ref, bb_ref, et_ref, hn_ref, t_ref, scr)


def _mlp_last_body(h_ref, a_ref, w1_ref, b1_ref, w2_ref, b2_ref,
                   g_ref, bb_ref, et_ref, hn_ref, scr):
    _mlp_body(False, h_ref, a_ref, w1_ref, b1_ref, w2_ref, b2_ref,
              g_ref, bb_ref, et_ref, hn_ref, None, scr)


_mlp_mid = pl.pallas_call(
    _mlp_mid_body,
    out_shape=[jax.ShapeDtypeStruct((_N, _HID), jnp.float32),
               jax.ShapeDtypeStruct((4, _N, _HID), jnp.float32)],
    scratch_shapes=[pltpu.VMEM((_N, _HID), jnp.float32)],
)

_mlp_last = pl.pallas_call(
    _mlp_last_body,
    out_shape=[jax.ShapeDtypeStruct((_N, _HID), jnp.float32)],
    scratch_shapes=[pltpu.VMEM((_N, _HID), jnp.float32)],
)


def _readout_body(h_ref, batch_ref, w1_ref, b1_ref, w2_ref, b2_ref, o_ref,
                  maxp_scr, scr_g, scr_o):
    h = h_ref[...]
    batch = batch_ref[...]
    gids = lax.broadcasted_iota(jnp.int32, (_N, _G), 1)
    onef = (batch == gids).astype(jnp.float32)
    dn = (((0,), (0,)), ((), ()))
    sums = _dot3(onef, h, scr_g, dn)
    ones = jnp.ones((_N, 1), jnp.float32)
    cnt = lax.dot_general(onef, ones, dn, preferred_element_type=jnp.float32)
    meanp = sums / jnp.maximum(cnt, 1.0)

    def mx(g, carry):
        masked = jnp.where(batch == g, h, -3e38)
        maxp_scr[pl.ds(g, 1), :] = jnp.max(masked, axis=0, keepdims=True)
        return carry

    lax.fori_loop(0, _G, mx, 0)
    maxp = jnp.where(cnt > 0.0, maxp_scr[...], 0.0)
    gemb = jnp.concatenate([meanp, maxp], axis=1)
    z = jnp.maximum(_dot3(gemb, w1_ref[...], scr_g) + b1_ref[...], 0.0)
    o_ref[...] = _dot3(z, w2_ref[...], scr_o) + b2_ref[...]


_readout = pl.pallas_call(
    _readout_body,
    out_shape=jax.ShapeDtypeStruct((_G, 10), jnp.float32),
    scratch_shapes=[pltpu.VMEM((_G, _HID), jnp.float32),
                    pltpu.VMEM((_G, _HID), jnp.float32),
                    pltpu.VMEM((_G, 10), jnp.float32)],
)


def kernel(x, edge_index, edge_attr, batch, x_lin_W, x_lin_b, edge_table,
           W1, b1, W2, b2, bn_g, bn_b, lin1_W, lin1_b, lin2_W, lin2_b):
    src = edge_index[0].astype(jnp.int32)
    dst = edge_index[1].astype(jnp.int32)
    gidx = edge_attr.astype(jnp.int32) * _N + src
    zeros = jnp.zeros((_RPT, _HID), jnp.float32)
    r = lambda a: a.reshape(1, -1)

    h, t = _encode(x, x_lin_W, r(x_lin_b), edge_table)
    for l in range(3):
        aggs = _edge_pass(t.reshape(4 * _N, _HID), gidx, dst, zeros)
        aggs = aggs[0, :_N] + aggs[1, :_N]
        if l < 2:
            h, t = _mlp_mid(h, aggs, W1[l], r(b1[l]), W2[l], r(b2[l]),
                            r(bn_g[l]), r(bn_b[l]), edge_table)
        else:
            (h,) = _mlp_last(h, aggs, W1[l], r(b1[l]), W2[l], r(b2[l]),
                             r(bn_g[l]), r(bn_b[l]), edge_table)
    return _readout(h, batch.astype(jnp.int32).reshape(-1, 1), lin1_W,
                    r(lin1_b), lin2_W, r(lin2_b))
